# initial kernel scaffold (unmeasured)
import jax
import jax.numpy as jnp
from jax import lax
from jax.experimental import pallas as pl
from jax.experimental.pallas import tpu as pltpu

N_DEV = 4
M = 4096
D = 4096
MB = M // N_DEV
CH = 256
NCH = MB // CH


def kernel(partial, resid, gamma):
    x = partial.reshape(M, D)
    g = gamma.reshape(1, D)

    def body(x_ref, resid_ref, gamma_ref, out_ref,
             rs_recv, send_buf, va, vb, vr, vc,
             rs_sems, ag_sems, cp_sems):
        my = lax.axis_index("i")
        left = (my - 1) % N_DEV
        right = (my + 1) % N_DEV

        barrier_sem = pltpu.get_barrier_semaphore()
        for nbr in (left, right):
            pl.semaphore_signal(barrier_sem, inc=1, device_id=(nbr,),
                                device_id_type=pl.DeviceIdType.MESH)
        pl.semaphore_wait(barrier_sem, 2)

        def blk_rows(b, c=None):
            if c is None:
                return pl.ds(b * MB, MB)
            return pl.ds(b * MB + c * CH, CH)

        def load(src, dst, slot):
            cp = pltpu.make_async_copy(src, dst, cp_sems.at[slot])
            cp.start()
            return cp

        for s in range(3):
            src = (x_ref.at[blk_rows(my % N_DEV), :] if s == 0
                   else send_buf.at[s - 1])
            rdma = pltpu.make_async_remote_copy(
                src_ref=src,
                dst_ref=rs_recv.at[s],
                send_sem=rs_sems.at[0, s],
                recv_sem=rs_sems.at[1, s],
                device_id=(right,),
                device_id_type=pl.DeviceIdType.MESH,
            )
            rdma.start()
            rdma.wait()
            if s < 2:
                b = (my - s - 1) % N_DEV
                for c in range(NCH):
                    cp1 = load(x_ref.at[blk_rows(b, c), :], va, 0)
                    cp2 = load(rs_recv.at[s, pl.ds(c * CH, CH), :], vb, 1)
                    cp1.wait()
                    cp2.wait()
                    vc[...] = va[...] + vb[...]
                    cpo = pltpu.make_async_copy(
                        vc, send_buf.at[s, pl.ds(c * CH, CH), :],
                        cp_sems.at[2])
                    cpo.start()
                    cpo.wait()

        o = (my + 1) % N_DEV
        for c in range(NCH):
            cp1 = load(x_ref.at[blk_rows(o, c), :], va, 0)
            cp2 = load(rs_recv.at[2, pl.ds(c * CH, CH), :], vb, 1)
            cp3 = load(resid_ref.at[blk_rows(o, c), :], vr, 3)
            cp1.wait()
            cp2.wait()
            cp3.wait()
            y = va[...] + vb[...] + vr[...]
            ms = jnp.sum(y * y, axis=1, keepdims=True) * (1.0 / D)
            vc[...] = y * lax.rsqrt(ms + 1e-6) * gamma_ref[...]
            cpo = pltpu.make_async_copy(vc, out_ref.at[blk_rows(o, c), :],
                                        cp_sems.at[2])
            cpo.start()
            cpo.wait()

        for s in range(3):
            q = (o - s) % N_DEV
            rdma = pltpu.make_async_remote_copy(
                src_ref=out_ref.at[blk_rows(q), :],
                dst_ref=out_ref.at[blk_rows(q), :],
                send_sem=ag_sems.at[0, s],
                recv_sem=ag_sems.at[1, s],
                device_id=(right,),
                device_id_type=pl.DeviceIdType.MESH,
            )
            rdma.start()
            rdma.wait()

    return pl.pallas_call(
        body,
        out_shape=jax.ShapeDtypeStruct((M, D), jnp.float32),
        in_specs=[
            pl.BlockSpec(memory_space=pl.ANY),
            pl.BlockSpec(memory_space=pl.ANY),
            pl.BlockSpec(memory_space=pltpu.VMEM),
        ],
        out_specs=pl.BlockSpec(memory_space=pl.ANY),
        scratch_shapes=[
            pltpu.MemorySpace.HBM((3, MB, D), jnp.float32),
            pltpu.MemorySpace.HBM((2, MB, D), jnp.float32),
            pltpu.VMEM((CH, D), jnp.float32),
            pltpu.VMEM((CH, D), jnp.float32),
            pltpu.VMEM((CH, D), jnp.float32),
            pltpu.VMEM((CH, D), jnp.float32),
            pltpu.SemaphoreType.DMA((2, 3)),
            pltpu.SemaphoreType.DMA((2, 3)),
            pltpu.SemaphoreType.DMA((4,)),
        ],
        compiler_params=pltpu.CompilerParams(collective_id=0),
    )(x, resid, g)


# baseline (device time: 1215383 ns/iter reference)
import jax
import jax.numpy as jnp
from jax import lax
from jax.experimental import pallas as pl
from jax.experimental.pallas import tpu as pltpu

N_DEV = 4
M = 4096
D = 4096
MB = M // N_DEV
CH = 256
NCH = MB // CH


def kernel(partial, resid, gamma):
    x = partial.reshape(M, D)
    g = gamma.reshape(1, D)

    def body(x_ref, resid_ref, gamma_ref, out_ref, rs_recv, send_buf,
             va, vb, vr, vc,
             rs_sems, ag_sems, cp_sems):
        my = lax.axis_index("i")
        left = (my - 1) % N_DEV
        right = (my + 1) % N_DEV

        barrier_sem = pltpu.get_barrier_semaphore()
        for nbr in (left, right):
            pl.semaphore_signal(barrier_sem, inc=1, device_id=(nbr,),
                                device_id_type=pl.DeviceIdType.MESH)
        pl.semaphore_wait(barrier_sem, 2)

        def blk_rows(b, c=None):
            if c is None:
                return pl.ds(b * MB, MB)
            return pl.ds(b * MB + c * CH, CH)

        def load(src, dst, slot):
            cp = pltpu.make_async_copy(src, dst, cp_sems.at[slot])
            cp.start()
            return cp

        for s in range(3):
            src = (x_ref.at[blk_rows(my % N_DEV), :] if s == 0
                   else send_buf.at[s - 1])
            rdma = pltpu.make_async_remote_copy(
                src_ref=src,
                dst_ref=rs_recv.at[s],
                send_sem=rs_sems.at[0, s],
                recv_sem=rs_sems.at[1, s],
                device_id=(right,),
                device_id_type=pl.DeviceIdType.MESH,
            )
            rdma.start()
            rdma.wait()
            if s < 2:
                b = (my - s - 1) % N_DEV
                for c in range(NCH):
                    cp1 = load(x_ref.at[blk_rows(b, c), :], va, 0)
                    cp2 = load(rs_recv.at[s, pl.ds(c * CH, CH), :], vb, 1)
                    cp1.wait()
                    cp2.wait()
                    vc[...] = va[...] + vb[...]
                    cpo = pltpu.make_async_copy(
                        vc, send_buf.at[s, pl.ds(c * CH, CH), :],
                        cp_sems.at[2])
                    cpo.start()
                    cpo.wait()

        o = (my + 1) % N_DEV
        for c in range(NCH):
            cp1 = load(x_ref.at[blk_rows(o, c), :], va, 0)
            cp2 = load(rs_recv.at[2, pl.ds(c * CH, CH), :], vb, 1)
            cp3 = load(resid_ref.at[blk_rows(o, c), :], vr, 3)
            cp1.wait()
            cp2.wait()
            cp3.wait()
            y = va[...] + vb[...] + vr[...]
            ms = jnp.sum(y * y, axis=1, keepdims=True) * (1.0 / D)
            vc[...] = y * lax.rsqrt(ms + 1e-6) * gamma_ref[...]
            cpo = pltpu.make_async_copy(vc, out_ref.at[blk_rows(o, c), :],
                                        cp_sems.at[2])
            cpo.start()
            cpo.wait()

        for s in range(3):
            q = (o - s) % N_DEV
            rdma = pltpu.make_async_remote_copy(
                src_ref=out_ref.at[blk_rows(q), :],
                dst_ref=out_ref.at[blk_rows(q), :],
                send_sem=ag_sems.at[0, s],
                recv_sem=ag_sems.at[1, s],
                device_id=(right,),
                device_id_type=pl.DeviceIdType.MESH,
            )
            rdma.start()
            rdma.wait()

    out, _, _ = pl.pallas_call(
        body,
        out_shape=[
            jax.ShapeDtypeStruct((M, D), jnp.float32),
            jax.ShapeDtypeStruct((3, MB, D), jnp.float32),
            jax.ShapeDtypeStruct((2, MB, D), jnp.float32),
        ],
        in_specs=[
            pl.BlockSpec(memory_space=pl.ANY),
            pl.BlockSpec(memory_space=pl.ANY),
            pl.BlockSpec(memory_space=pltpu.VMEM),
        ],
        out_specs=[
            pl.BlockSpec(memory_space=pl.ANY),
            pl.BlockSpec(memory_space=pl.ANY),
            pl.BlockSpec(memory_space=pl.ANY),
        ],
        scratch_shapes=[
            pltpu.VMEM((CH, D), jnp.float32),
            pltpu.VMEM((CH, D), jnp.float32),
            pltpu.VMEM((CH, D), jnp.float32),
            pltpu.VMEM((CH, D), jnp.float32),
            pltpu.SemaphoreType.DMA((2, 3)),
            pltpu.SemaphoreType.DMA((2, 3)),
            pltpu.SemaphoreType.DMA((4,)),
        ],
        compiler_params=pltpu.CompilerParams(collective_id=0),
    )(x, resid, g)
    return out
